# Initial kernel scaffold; baseline (speedup 1.0000x reference)
import jax
import jax.numpy as jnp
from jax.experimental import pallas as pl


def kernel(x, edge_index, edge_attr, batch, W1_0, W1_1, W1_2, b1, W2_0, W2_1, W2_2, b2, W3_0, W3_1, W3_2, b3, Wl, bl):
    raise NotImplementedError("write your pallas kernel here")


# speedup vs baseline: 3.3074x; 3.3074x over previous
import functools

import jax
import jax.numpy as jnp
from jax import lax
from jax.experimental import pallas as pl
from jax.experimental.pallas import tpu as pltpu
from jax.experimental.pallas import tpu_sc as plsc

NN = 10000
EE = 320000
GG = 16
NP = 10240
NC, NS = 2, 16
NW = NC * NS
EPW = EE // NW
CH = 80
NCHUNK = EPW // CH
RPT = NP // NS

F32 = jnp.float32


def _mesh():
    return plsc.VectorSubcoreMesh(
        core_axis_name="c", subcore_axis_name="s", num_cores=NC, num_subcores=NS
    )


def _wid():
    return lax.axis_index("s") * NC + lax.axis_index("c")


def _deg_body(src_h, dst_h, attr_h, out_h, sidx, dbuf, abuf, wbuf, zb, acc, sem):
    del sem
    cid = lax.axis_index("c")
    sid = lax.axis_index("s")
    wid = _wid()

    def zero(i, _):
        zb[pl.ds(i * 16, 16)] = jnp.zeros((16,), F32)
        return 0

    lax.fori_loop(0, RPT // 16, zero, 0)
    pltpu.sync_copy(zb, acc.at[pl.ds(sid * RPT, RPT)])
    plsc.subcore_barrier()

    def chunk(i, _):
        base = wid * EPW + i * CH
        pltpu.sync_copy(src_h.at[pl.ds(base, CH)], sidx)
        pltpu.sync_copy(dst_h.at[pl.ds(base, CH)], dbuf)
        pltpu.sync_copy(attr_h.at[pl.ds(base, CH)], abuf)
        for k in range(CH // 16):
            s16 = sidx[pl.ds(k * 16, 16)]
            d16 = dbuf[pl.ds(k * 16, 16)]
            a16 = abuf[pl.ds(k * 16, 16)]
            wbuf[pl.ds(k * 16, 16)] = jnp.where(s16 != d16, a16, 0.0)
        pltpu.sync_copy(wbuf, acc.at[sidx], add=True)
        return 0

    lax.fori_loop(0, NCHUNK, chunk, 0)
    plsc.subcore_barrier()
    pltpu.sync_copy(
        acc.at[pl.ds(sid * RPT, RPT)], out_h.at[pl.ds(cid * NP + sid * RPT, RPT)]
    )


@functools.cache
def _deg_kernel():
    return pl.kernel(
        _deg_body,
        out_type=jax.ShapeDtypeStruct((2 * NP,), F32),
        mesh=_mesh(),
        compiler_params=pltpu.CompilerParams(use_tc_tiling_on_sc=False),
        scratch_types=[
            pltpu.VMEM((CH,), jnp.int32),
            pltpu.VMEM((CH,), jnp.int32),
            pltpu.VMEM((CH,), F32),
            pltpu.VMEM((CH,), F32),
            pltpu.VMEM((RPT,), F32),
            pltpu.VMEM_SHARED((NP,), F32),
            pltpu.SemaphoreType.DMA,
        ],
    )


def _norm_body(src_h, dst_h, attr_h, dis_h, out_h, sidx, dbuf, abuf, dsb, ddb, nbuf, sem):
    wid = _wid()

    def chunk(i, _):
        base = wid * EPW + i * CH
        pltpu.sync_copy(src_h.at[pl.ds(base, CH)], sidx)
        pltpu.sync_copy(dst_h.at[pl.ds(base, CH)], dbuf)
        pltpu.sync_copy(attr_h.at[pl.ds(base, CH)], abuf)
        pltpu.async_copy(dis_h.at[sidx], dsb, sem).wait()
        pltpu.async_copy(dis_h.at[dbuf], ddb, sem).wait()
        for k in range(CH // 16):
            s16 = sidx[pl.ds(k * 16, 16)]
            d16 = dbuf[pl.ds(k * 16, 16)]
            a16 = abuf[pl.ds(k * 16, 16)]
            w16 = jnp.where(s16 != d16, a16, 0.0)
            nbuf[pl.ds(k * 16, 16)] = -(
                dsb[pl.ds(k * 16, 16)] * w16 * ddb[pl.ds(k * 16, 16)]
            )
        pltpu.sync_copy(nbuf, out_h.at[pl.ds(base, CH)])
        return 0

    lax.fori_loop(0, NCHUNK, chunk, 0)


@functools.cache
def _norm_kernel():
    return pl.kernel(
        _norm_body,
        out_type=jax.ShapeDtypeStruct((EE,), F32),
        mesh=_mesh(),
        compiler_params=pltpu.CompilerParams(use_tc_tiling_on_sc=False),
        scratch_types=[
            pltpu.VMEM((CH,), jnp.int32),
            pltpu.VMEM((CH,), jnp.int32),
            pltpu.VMEM((CH,), F32),
            pltpu.VMEM((CH,), F32),
            pltpu.VMEM((CH,), F32),
            pltpu.VMEM((CH,), F32),
            pltpu.SemaphoreType.DMA,
        ],
    )


@functools.cache
def _make_prop(D):
    JV = D // 16

    def body(y_h, src_h, dst_h, nrm_h, out_h, sidx, didx, nrmb, rows, zb, acc, sem):
        cid = lax.axis_index("c")
        sid = lax.axis_index("s")
        wid = _wid()

        def zero(r, _):
            for j in range(JV):
                zb[r, pl.ds(j * 16, 16)] = jnp.zeros((16,), F32)
            return 0

        lax.fori_loop(0, RPT, zero, 0)
        pltpu.sync_copy(zb, acc.at[pl.ds(sid * RPT, RPT)])
        plsc.subcore_barrier()

        def chunk(i, _):
            base = wid * EPW + i * CH
            pltpu.sync_copy(src_h.at[pl.ds(base, CH)], sidx)
            pltpu.sync_copy(dst_h.at[pl.ds(base, CH)], didx)
            pltpu.sync_copy(nrm_h.at[pl.ds(base, CH)], nrmb)
            pltpu.async_copy(y_h.at[sidx], rows, sem).wait()

            def scale(k, _):
                n16 = nrmb[pl.ds(k * 16, 16)]
                for e in range(16):
                    s = n16[e]
                    r = k * 16 + e
                    for j in range(JV):
                        rows[r, pl.ds(j * 16, 16)] = rows[r, pl.ds(j * 16, 16)] * s
                return 0

            lax.fori_loop(0, CH // 16, scale, 0)
            pltpu.sync_copy(rows, acc.at[didx], add=True)
            return 0

        lax.fori_loop(0, NCHUNK, chunk, 0)
        plsc.subcore_barrier()
        pltpu.sync_copy(
            acc.at[pl.ds(sid * RPT, RPT)],
            out_h.at[pl.ds(cid * NP + sid * RPT, RPT)],
        )

    return pl.kernel(
        body,
        out_type=jax.ShapeDtypeStruct((2 * NP, D), F32),
        mesh=_mesh(),
        compiler_params=pltpu.CompilerParams(use_tc_tiling_on_sc=False),
        scratch_types=[
            pltpu.VMEM((CH,), jnp.int32),
            pltpu.VMEM((CH,), jnp.int32),
            pltpu.VMEM((CH,), F32),
            pltpu.VMEM((CH, D), F32),
            pltpu.VMEM((RPT, D), F32),
            pltpu.VMEM_SHARED((NP, D), F32),
            pltpu.SemaphoreType.DMA,
        ],
    )


BR = 2048
NBLK = NP // BR


def _row_spec(width):
    return pl.BlockSpec((BR, width), lambda i: (i, 0))


def _row_spec_off(width, off_blocks):
    return pl.BlockSpec((BR, width), lambda i: (i + off_blocks, 0))


def _full_spec(r, c):
    return pl.BlockSpec((r, c), lambda i: (0, 0))


def _l1pre_body(x_r, degp_r, w1_r, w2_r, w02_r, b_r, y1_r, y2_r, a0_r, dis_r):
    xb = x_r[...]
    y1_r[...] = jnp.dot(xb, w1_r[...], preferred_element_type=F32)
    y2_r[...] = jnp.dot(xb, w2_r[...], preferred_element_type=F32)
    a0_r[...] = jnp.dot(xb, w02_r[...], preferred_element_type=F32) + b_r[...]
    deg = degp_r[0:1, :] + degp_r[1:2, :]
    dis_r[...] = jnp.where(deg > 0, lax.rsqrt(jnp.where(deg > 0, deg, 1.0)), 0.0)


def _tc_l1pre(xp, degp, w1, w2, w02, b1):
    return pl.pallas_call(
        _l1pre_body,
        grid=(NBLK,),
        in_specs=[
            _row_spec(128),
            pl.BlockSpec((2, BR), lambda i: (0, i)),
            _full_spec(128, 64),
            _full_spec(128, 64),
            _full_spec(128, 64),
            _full_spec(1, 64),
        ],
        out_specs=[
            _row_spec(64),
            _row_spec(64),
            _row_spec(64),
            pl.BlockSpec((1, BR), lambda i: (0, i)),
        ],
        out_shape=[
            jax.ShapeDtypeStruct((NP, 64), F32),
            jax.ShapeDtypeStruct((NP, 64), F32),
            jax.ShapeDtypeStruct((NP, 64), F32),
            jax.ShapeDtypeStruct((1, NP), F32),
        ],
    )(xp, degp, w1, w2, w02, b1)


def _comb_body(pa_r, pb_r, c_r):
    c_r[...] = pa_r[...] + pb_r[...]


def _tc_comb(pp):
    return pl.pallas_call(
        _comb_body,
        grid=(NBLK,),
        in_specs=[_row_spec(64), _row_spec_off(64, NBLK)],
        out_specs=_row_spec(64),
        out_shape=jax.ShapeDtypeStruct((NP, 64), F32),
    )(pp, pp)


def _h1_body(a0_r, pa_r, pb_r, qa_r, qb_r, h1_r):
    h1_r[...] = jnp.maximum(
        a0_r[...] + pa_r[...] + pb_r[...] + 2.0 * (qa_r[...] + qb_r[...]), 0.0
    )


def _tc_h1(a0, p1p, qp):
    return pl.pallas_call(
        _h1_body,
        grid=(NBLK,),
        in_specs=[
            _row_spec(64),
            _row_spec(64),
            _row_spec_off(64, NBLK),
            _row_spec(64),
            _row_spec_off(64, NBLK),
        ],
        out_specs=_row_spec(64),
        out_shape=jax.ShapeDtypeStruct((NP, 64), F32),
    )(a0, p1p, p1p, qp, qp)


def _make_mid(dout):
    def body(h_r, ta_r, tb_r, w02_r, w1_r, b_r, t1c_r, z_r):
        t1 = ta_r[...] + tb_r[...]
        t1c_r[...] = t1
        z_r[...] = (
            jnp.dot(h_r[...], w02_r[...], preferred_element_type=F32)
            + jnp.dot(t1, w1_r[...], preferred_element_type=F32)
            + b_r[...]
        )

    def call(h, t1p, w02, w1, b):
        return pl.pallas_call(
            body,
            grid=(NBLK,),
            in_specs=[
                _row_spec(64),
                _row_spec(64),
                _row_spec_off(64, NBLK),
                _full_spec(64, dout),
                _full_spec(64, dout),
                _full_spec(1, dout),
            ],
            out_specs=[_row_spec(64), _row_spec(dout)],
            out_shape=[
                jax.ShapeDtypeStruct((NP, 64), F32),
                jax.ShapeDtypeStruct((NP, dout), F32),
            ],
        )(h, t1p, t1p, w02, w1, b)

    return call


_tc_mid64 = _make_mid(64)
_tc_mid128 = _make_mid(128)


def _make_post(dout):
    def body(z_r, ta_r, tb_r, w2_r, h_r):
        t2 = ta_r[...] + tb_r[...]
        h_r[...] = jnp.maximum(
            z_r[...] + 2.0 * jnp.dot(t2, w2_r[...], preferred_element_type=F32), 0.0
        )

    def call(z, t2p, w2):
        return pl.pallas_call(
            body,
            grid=(NBLK,),
            in_specs=[
                _row_spec(dout),
                _row_spec(64),
                _row_spec_off(64, NBLK),
                _full_spec(64, dout),
            ],
            out_specs=_row_spec(dout),
            out_shape=jax.ShapeDtypeStruct((NP, dout), F32),
        )(z, t2p, t2p, w2)

    return call


_tc_post64 = _make_post(64)
_tc_post128 = _make_post(128)


def _final_body(h3_r, batch_r, wl_r, bl_r, out_r, hg_r):
    gi = lax.broadcasted_iota(jnp.int32, (GG, NP), 0)
    oh = (batch_r[...] == gi).astype(F32)
    sums = jnp.dot(oh, h3_r[...], preferred_element_type=F32)
    cnt = jnp.sum(oh, axis=1, keepdims=True)
    hg = sums / jnp.maximum(cnt, 1.0)
    hg_r[...] = hg
    out_r[...] = jnp.dot(hg, wl_r[...], preferred_element_type=F32) + bl_r[...]


def _tc_final(h3, batchp, wl, bl):
    return pl.pallas_call(
        _final_body,
        out_shape=[
            jax.ShapeDtypeStruct((GG, 10), F32),
            jax.ShapeDtypeStruct((GG, 128), F32),
        ],
    )(h3, batchp, wl, bl)


@jax.jit
def kernel(x, edge_index, edge_attr, batch, W1_0, W1_1, W1_2, b1, W2_0, W2_1, W2_2,
           b2, W3_0, W3_1, W3_2, b3, Wl, bl):
    src = edge_index[0]
    dst = edge_index[1]

    xp = jnp.pad(x, ((0, NP - NN), (0, 0)))
    batchp = jnp.pad(batch, (0, NP - NN), constant_values=-1).reshape(1, NP)

    w1_02 = W1_0 - W1_2
    w2_02 = W2_0 - W2_2
    w3_02 = W3_0 - W3_2

    prop64 = _make_prop(64)

    degp = _deg_kernel()(src, dst, edge_attr).reshape(2, NP)
    y1, y2, a0, dis = _tc_l1pre(xp, degp, W1_1, W1_2, w1_02, b1.reshape(1, 64))
    norm = _norm_kernel()(src, dst, edge_attr, dis.reshape(NP))

    p1p = prop64(y1, src, dst, norm)
    p2p = prop64(y2, src, dst, norm)
    p2c = _tc_comb(p2p)
    qp = prop64(p2c, src, dst, norm)
    h1 = _tc_h1(a0, p1p, qp)

    t1p = prop64(h1, src, dst, norm)
    t1c, z2 = _tc_mid64(h1, t1p, w2_02, W2_1, b2.reshape(1, 64))
    t2p = prop64(t1c, src, dst, norm)
    h2 = _tc_post64(z2, t2p, W2_2)

    u1p = prop64(h2, src, dst, norm)
    u1c, z3 = _tc_mid128(h2, u1p, w3_02, W3_1, b3.reshape(1, 128))
    u2p = prop64(u1c, src, dst, norm)
    h3 = _tc_post128(z3, u2p, W3_2)

    out, hg = _tc_final(h3, batchp, Wl, bl.reshape(1, 10))
    return (out, hg)


# speedup vs baseline: 9.3565x; 2.8290x over previous
import functools

import jax
import jax.numpy as jnp
from jax import lax
from jax.experimental import pallas as pl
from jax.experimental.pallas import tpu as pltpu
from jax.experimental.pallas import tpu_sc as plsc

NN = 10000
EE = 320000
GG = 16
NP = 10240
NC, NS = 2, 16
NW = NC * NS
EPW = EE // NW
CH = 400
NCHUNK = EPW // CH
RPT = NP // NS
ZR = 128

F32 = jnp.float32


def _mesh():
    return plsc.VectorSubcoreMesh(
        core_axis_name="c", subcore_axis_name="s", num_cores=NC, num_subcores=NS
    )


def _wid():
    return lax.axis_index("s") * NC + lax.axis_index("c")


def _deg_body(src_h, dst_h, attr_h, out_h, esrc, edst, eatt, wbuf, zb, acc, sem, ssem):
    cid = lax.axis_index("c")
    sid = lax.axis_index("s")
    wid = _wid()

    pltpu.async_copy(src_h.at[pl.ds(wid, 1)], esrc, sem).wait()
    pltpu.async_copy(dst_h.at[pl.ds(wid, 1)], edst, sem).wait()
    pltpu.async_copy(attr_h.at[pl.ds(wid, 1)], eatt, sem).wait()

    def zero(i, _):
        zb[pl.ds(i * 16, 16)] = jnp.zeros((16,), F32)
        return 0

    lax.fori_loop(0, RPT // 16, zero, 0)
    pltpu.sync_copy(zb, acc.at[pl.ds(sid * RPT, RPT)])

    def wcomp(i, _):
        def sub(k, _):
            s16 = esrc[0, i, pl.ds(k * 16, 16)]
            d16 = edst[0, i, pl.ds(k * 16, 16)]
            a16 = eatt[0, i, pl.ds(k * 16, 16)]
            wbuf[i, pl.ds(k * 16, 16)] = jnp.where(s16 != d16, a16, 0.0)
            return 0

        lax.fori_loop(0, CH // 16, sub, 0)
        return 0

    lax.fori_loop(0, NCHUNK, wcomp, 0)
    plsc.subcore_barrier()

    descs = []
    for i in range(NCHUNK):
        descs.append(
            pltpu.async_copy(wbuf.at[i], acc.at[esrc.at[0, i]], ssem, add=True)
        )
        if i >= 12:
            descs[i - 12].wait()
    for d in descs[NCHUNK - 12:]:
        d.wait()
    plsc.subcore_barrier()
    pltpu.sync_copy(
        acc.at[pl.ds(sid * RPT, RPT)], out_h.at[pl.ds(cid * NP + sid * RPT, RPT)]
    )


@functools.cache
def _deg_kernel():
    return pl.kernel(
        _deg_body,
        out_type=jax.ShapeDtypeStruct((2 * NP,), F32),
        mesh=_mesh(),
        compiler_params=pltpu.CompilerParams(use_tc_tiling_on_sc=False),
        scratch_types=[
            pltpu.VMEM((1, NCHUNK, CH), jnp.int32),
            pltpu.VMEM((1, NCHUNK, CH), jnp.int32),
            pltpu.VMEM((1, NCHUNK, CH), F32),
            pltpu.VMEM((NCHUNK, CH), F32),
            pltpu.VMEM((RPT,), F32),
            pltpu.VMEM_SHARED((NP,), F32),
            pltpu.SemaphoreType.DMA,
            pltpu.SemaphoreType.DMA,
        ],
    )


def _norm_body(src_h, dst_h, attr_h, dis_h, out_h, esrc, edst, eatt, dsb, ddb, nbuf, sem, gsem):
    wid = _wid()

    pltpu.async_copy(src_h.at[pl.ds(wid, 1)], esrc, sem).wait()
    pltpu.async_copy(dst_h.at[pl.ds(wid, 1)], edst, sem).wait()
    pltpu.async_copy(attr_h.at[pl.ds(wid, 1)], eatt, sem).wait()

    gd = [None, None]
    gd[0] = (
        pltpu.async_copy(dis_h.at[esrc.at[0, 0]], dsb.at[0], gsem),
        pltpu.async_copy(dis_h.at[edst.at[0, 0]], ddb.at[0], gsem),
    )
    for i in range(NCHUNK):
        cur = i % 2
        nxt = 1 - cur
        for d in gd[cur]:
            d.wait()
        if i + 1 < NCHUNK:
            gd[nxt] = (
                pltpu.async_copy(dis_h.at[esrc.at[0, i + 1]], dsb.at[nxt], gsem),
                pltpu.async_copy(dis_h.at[edst.at[0, i + 1]], ddb.at[nxt], gsem),
            )

        def sub(k, _):
            s16 = esrc[0, i, pl.ds(k * 16, 16)]
            d16 = edst[0, i, pl.ds(k * 16, 16)]
            a16 = eatt[0, i, pl.ds(k * 16, 16)]
            w16 = jnp.where(s16 != d16, a16, 0.0)
            nbuf[0, i, pl.ds(k * 16, 16)] = -(
                dsb[cur, pl.ds(k * 16, 16)] * w16 * ddb[cur, pl.ds(k * 16, 16)]
            )
            return 0

        lax.fori_loop(0, CH // 16, sub, 0)
    pltpu.sync_copy(nbuf, out_h.at[pl.ds(wid, 1)])


@functools.cache
def _norm_kernel():
    return pl.kernel(
        _norm_body,
        out_type=jax.ShapeDtypeStruct((NW, NCHUNK, CH), F32),
        mesh=_mesh(),
        compiler_params=pltpu.CompilerParams(use_tc_tiling_on_sc=False),
        scratch_types=[
            pltpu.VMEM((1, NCHUNK, CH), jnp.int32),
            pltpu.VMEM((1, NCHUNK, CH), jnp.int32),
            pltpu.VMEM((1, NCHUNK, CH), F32),
            pltpu.VMEM((2, CH), F32),
            pltpu.VMEM((2, CH), F32),
            pltpu.VMEM((1, NCHUNK, CH), F32),
            pltpu.SemaphoreType.DMA,
            pltpu.SemaphoreType.DMA,
        ],
    )


@functools.cache
def _make_prop(D):
    JV = D // 16

    def body(y_h, src_h, dst_h, nrm_h, out_h, esrc, edst, enrm, rows, zb, acc,
             sem, gsA, gsB, ssA, ssB):
        cid = lax.axis_index("c")
        sid = lax.axis_index("s")
        wid = _wid()
        gsem = [gsA, gsB]
        ssem = [ssA, ssB]

        pltpu.async_copy(src_h.at[pl.ds(wid, 1)], esrc, sem).wait()
        pltpu.async_copy(dst_h.at[pl.ds(wid, 1)], edst, sem).wait()
        pltpu.async_copy(nrm_h.at[pl.ds(wid, 1)], enrm, sem).wait()

        def zero(r, _):
            for j in range(JV):
                zb[r, pl.ds(j * 16, 16)] = jnp.zeros((16,), F32)
            return 0

        lax.fori_loop(0, ZR, zero, 0)
        for k in range(RPT // ZR):
            pltpu.sync_copy(zb, acc.at[pl.ds(sid * RPT + k * ZR, ZR)])
        plsc.subcore_barrier()

        def gather(i, b):
            return pltpu.async_copy(y_h.at[esrc.at[0, i]], rows.at[b], gsem[b])

        def scatter(i, b):
            return pltpu.async_copy(rows.at[b], acc.at[edst.at[0, i]], ssem[b], add=True)

        gd = [None, None]
        sd = [None, None]
        gd[0] = gather(0, 0)
        for i in range(NCHUNK):
            cur = i % 2
            nxt = 1 - cur
            gd[cur].wait()

            def scale(k, _):
                n16 = enrm[0, i, pl.ds(k * 16, 16)]
                for e in range(16):
                    s = n16[e]
                    r = k * 16 + e
                    for j in range(JV):
                        rows[cur, r, pl.ds(j * 16, 16)] = (
                            rows[cur, r, pl.ds(j * 16, 16)] * s
                        )
                return 0

            lax.fori_loop(0, CH // 16, scale, 0)
            if i + 1 < NCHUNK:
                if sd[nxt] is not None:
                    sd[nxt].wait()
                gd[nxt] = gather(i + 1, nxt)
            sd[cur] = scatter(i, cur)
        for d in sd:
            if d is not None:
                d.wait()
        plsc.subcore_barrier()
        for k in range(RPT // ZR):
            pltpu.sync_copy(
                acc.at[pl.ds(sid * RPT + k * ZR, ZR)],
                out_h.at[pl.ds(cid * NP + sid * RPT + k * ZR, ZR)],
            )

    return pl.kernel(
        body,
        out_type=jax.ShapeDtypeStruct((2 * NP, D), F32),
        mesh=_mesh(),
        compiler_params=pltpu.CompilerParams(use_tc_tiling_on_sc=False),
        scratch_types=[
            pltpu.VMEM((1, NCHUNK, CH), jnp.int32),
            pltpu.VMEM((1, NCHUNK, CH), jnp.int32),
            pltpu.VMEM((1, NCHUNK, CH), F32),
            pltpu.VMEM((2, CH, D), F32),
            pltpu.VMEM((ZR, D), F32),
            pltpu.VMEM_SHARED((NP, D), F32),
            pltpu.SemaphoreType.DMA,
            pltpu.SemaphoreType.DMA,
            pltpu.SemaphoreType.DMA,
            pltpu.SemaphoreType.DMA,
            pltpu.SemaphoreType.DMA,
        ],
    )


BR = 2048
NBLK = NP // BR


def _row_spec(width):
    return pl.BlockSpec((BR, width), lambda i: (i, 0))


def _row_spec_off(width, off_blocks):
    return pl.BlockSpec((BR, width), lambda i: (i + off_blocks, 0))


def _full_spec(r, c):
    return pl.BlockSpec((r, c), lambda i: (0, 0))


def _l1pre_body(x_r, degp_r, w1_r, w2_r, w02_r, b_r, y1_r, y2_r, a0_r, dis_r):
    xb = x_r[...]
    y1_r[...] = jnp.dot(xb, w1_r[...], preferred_element_type=F32)
    y2_r[...] = jnp.dot(xb, w2_r[...], preferred_element_type=F32)
    a0_r[...] = jnp.dot(xb, w02_r[...], preferred_element_type=F32) + b_r[...]
    deg = degp_r[0:1, :] + degp_r[1:2, :]
    dis_r[...] = jnp.where(deg > 0, lax.rsqrt(jnp.where(deg > 0, deg, 1.0)), 0.0)


def _tc_l1pre(xp, degp, w1, w2, w02, b1):
    return pl.pallas_call(
        _l1pre_body,
        grid=(NBLK,),
        in_specs=[
            _row_spec(128),
            pl.BlockSpec((2, BR), lambda i: (0, i)),
            _full_spec(128, 64),
            _full_spec(128, 64),
            _full_spec(128, 64),
            _full_spec(1, 64),
        ],
        out_specs=[
            _row_spec(64),
            _row_spec(64),
            _row_spec(64),
            pl.BlockSpec((1, BR), lambda i: (0, i)),
        ],
        out_shape=[
            jax.ShapeDtypeStruct((NP, 64), F32),
            jax.ShapeDtypeStruct((NP, 64), F32),
            jax.ShapeDtypeStruct((NP, 64), F32),
            jax.ShapeDtypeStruct((1, NP), F32),
        ],
    )(xp, degp, w1, w2, w02, b1)


def _comb_body(pa_r, pb_r, c_r):
    c_r[...] = pa_r[...] + pb_r[...]


def _tc_comb(pp):
    return pl.pallas_call(
        _comb_body,
        grid=(NBLK,),
        in_specs=[_row_spec(64), _row_spec_off(64, NBLK)],
        out_specs=_row_spec(64),
        out_shape=jax.ShapeDtypeStruct((NP, 64), F32),
    )(pp, pp)


def _h1_body(a0_r, pa_r, pb_r, qa_r, qb_r, h1_r):
    h1_r[...] = jnp.maximum(
        a0_r[...] + pa_r[...] + pb_r[...] + 2.0 * (qa_r[...] + qb_r[...]), 0.0
    )


def _tc_h1(a0, p1p, qp):
    return pl.pallas_call(
        _h1_body,
        grid=(NBLK,),
        in_specs=[
            _row_spec(64),
            _row_spec(64),
            _row_spec_off(64, NBLK),
            _row_spec(64),
            _row_spec_off(64, NBLK),
        ],
        out_specs=_row_spec(64),
        out_shape=jax.ShapeDtypeStruct((NP, 64), F32),
    )(a0, p1p, p1p, qp, qp)


def _make_mid(dout):
    def body(h_r, ta_r, tb_r, w02_r, w1_r, b_r, t1c_r, z_r):
        t1 = ta_r[...] + tb_r[...]
        t1c_r[...] = t1
        z_r[...] = (
            jnp.dot(h_r[...], w02_r[...], preferred_element_type=F32)
            + jnp.dot(t1, w1_r[...], preferred_element_type=F32)
            + b_r[...]
        )

    def call(h, t1p, w02, w1, b):
        return pl.pallas_call(
            body,
            grid=(NBLK,),
            in_specs=[
                _row_spec(64),
                _row_spec(64),
                _row_spec_off(64, NBLK),
                _full_spec(64, dout),
                _full_spec(64, dout),
                _full_spec(1, dout),
            ],
            out_specs=[_row_spec(64), _row_spec(dout)],
            out_shape=[
                jax.ShapeDtypeStruct((NP, 64), F32),
                jax.ShapeDtypeStruct((NP, dout), F32),
            ],
        )(h, t1p, t1p, w02, w1, b)

    return call


_tc_mid64 = _make_mid(64)
_tc_mid128 = _make_mid(128)


def _make_post(dout):
    def body(z_r, ta_r, tb_r, w2_r, h_r):
        t2 = ta_r[...] + tb_r[...]
        h_r[...] = jnp.maximum(
            z_r[...] + 2.0 * jnp.dot(t2, w2_r[...], preferred_element_type=F32), 0.0
        )

    def call(z, t2p, w2):
        return pl.pallas_call(
            body,
            grid=(NBLK,),
            in_specs=[
                _row_spec(dout),
                _row_spec(64),
                _row_spec_off(64, NBLK),
                _full_spec(64, dout),
            ],
            out_specs=_row_spec(dout),
            out_shape=jax.ShapeDtypeStruct((NP, dout), F32),
        )(z, t2p, t2p, w2)

    return call


_tc_post64 = _make_post(64)
_tc_post128 = _make_post(128)


def _final_body(h3_r, batch_r, wl_r, bl_r, out_r, hg_r):
    gi = lax.broadcasted_iota(jnp.int32, (GG, NP), 0)
    oh = (batch_r[...] == gi).astype(F32)
    sums = jnp.dot(oh, h3_r[...], preferred_element_type=F32)
    cnt = jnp.sum(oh, axis=1, keepdims=True)
    hg = sums / jnp.maximum(cnt, 1.0)
    hg_r[...] = hg
    out_r[...] = jnp.dot(hg, wl_r[...], preferred_element_type=F32) + bl_r[...]


def _tc_final(h3, batchp, wl, bl):
    return pl.pallas_call(
        _final_body,
        out_shape=[
            jax.ShapeDtypeStruct((GG, 10), F32),
            jax.ShapeDtypeStruct((GG, 128), F32),
        ],
    )(h3, batchp, wl, bl)


@jax.jit
def kernel(x, edge_index, edge_attr, batch, W1_0, W1_1, W1_2, b1, W2_0, W2_1, W2_2,
           b2, W3_0, W3_1, W3_2, b3, Wl, bl):
    src = edge_index[0].reshape(NW, NCHUNK, CH)
    dst = edge_index[1].reshape(NW, NCHUNK, CH)
    attr3 = edge_attr.reshape(NW, NCHUNK, CH)

    xp = jnp.pad(x, ((0, NP - NN), (0, 0)))
    batchp = jnp.pad(batch, (0, NP - NN), constant_values=-1).reshape(1, NP)

    w1_02 = W1_0 - W1_2
    w2_02 = W2_0 - W2_2
    w3_02 = W3_0 - W3_2

    prop64 = _make_prop(64)

    degp = _deg_kernel()(src, dst, attr3).reshape(2, NP)
    y1, y2, a0, dis = _tc_l1pre(xp, degp, W1_1, W1_2, w1_02, b1.reshape(1, 64))
    norm = _norm_kernel()(src, dst, attr3, dis.reshape(NP))

    p1p = prop64(y1, src, dst, norm)
    p2p = prop64(y2, src, dst, norm)
    p2c = _tc_comb(p2p)
    qp = prop64(p2c, src, dst, norm)
    h1 = _tc_h1(a0, p1p, qp)

    t1p = prop64(h1, src, dst, norm)
    t1c, z2 = _tc_mid64(h1, t1p, w2_02, W2_1, b2.reshape(1, 64))
    t2p = prop64(t1c, src, dst, norm)
    h2 = _tc_post64(z2, t2p, W2_2)

    u1p = prop64(h2, src, dst, norm)
    u1c, z3 = _tc_mid128(h2, u1p, w3_02, W3_1, b3.reshape(1, 128))
    u2p = prop64(u1c, src, dst, norm)
    h3 = _tc_post128(z3, u2p, W3_2)

    out, hg = _tc_final(h3, batchp, Wl, bl.reshape(1, 10))
    return (out, hg)


# speedup vs baseline: 10.4911x; 1.1213x over previous
import functools

import jax
import jax.numpy as jnp
from jax import lax
from jax.experimental import pallas as pl
from jax.experimental.pallas import tpu as pltpu
from jax.experimental.pallas import tpu_sc as plsc

NN = 10000
EE = 320000
GG = 16
NP = 10240
NC, NS = 2, 16
NW = NC * NS
EPW = EE // NW
CH = 400
NCHUNK = EPW // CH
RPT = NP // NS
ZR = 128

F32 = jnp.float32


def _mesh():
    return plsc.VectorSubcoreMesh(
        core_axis_name="c", subcore_axis_name="s", num_cores=NC, num_subcores=NS
    )


def _wid():
    return lax.axis_index("s") * NC + lax.axis_index("c")


def _deg_body(src_h, dst_h, attr_h, out_h, esrc, edst, eatt, wbuf, zb, acc, sem, ssem):
    cid = lax.axis_index("c")
    sid = lax.axis_index("s")
    wid = _wid()

    pltpu.async_copy(src_h.at[pl.ds(wid, 1)], esrc, sem).wait()
    pltpu.async_copy(dst_h.at[pl.ds(wid, 1)], edst, sem).wait()
    pltpu.async_copy(attr_h.at[pl.ds(wid, 1)], eatt, sem).wait()

    def zero(i, _):
        zb[pl.ds(i * 16, 16)] = jnp.zeros((16,), F32)
        return 0

    lax.fori_loop(0, RPT // 16, zero, 0)
    pltpu.sync_copy(zb, acc.at[pl.ds(sid * RPT, RPT)])

    def wcomp(i, _):
        def sub(k, _):
            s16 = esrc[0, i, pl.ds(k * 16, 16)]
            d16 = edst[0, i, pl.ds(k * 16, 16)]
            a16 = eatt[0, i, pl.ds(k * 16, 16)]
            wbuf[i, pl.ds(k * 16, 16)] = jnp.where(s16 != d16, a16, 0.0)
            return 0

        lax.fori_loop(0, CH // 16, sub, 0)
        return 0

    lax.fori_loop(0, NCHUNK, wcomp, 0)
    plsc.subcore_barrier()

    descs = []
    for i in range(NCHUNK):
        descs.append(
            pltpu.async_copy(wbuf.at[i], acc.at[esrc.at[0, i]], ssem, add=True)
        )
        if i >= 12:
            descs[i - 12].wait()
    for d in descs[NCHUNK - 12:]:
        d.wait()
    plsc.subcore_barrier()
    pltpu.sync_copy(
        acc.at[pl.ds(sid * RPT, RPT)], out_h.at[pl.ds(cid * NP + sid * RPT, RPT)]
    )


@functools.cache
def _deg_kernel():
    return pl.kernel(
        _deg_body,
        out_type=jax.ShapeDtypeStruct((2 * NP,), F32),
        mesh=_mesh(),
        compiler_params=pltpu.CompilerParams(use_tc_tiling_on_sc=False),
        scratch_types=[
            pltpu.VMEM((1, NCHUNK, CH), jnp.int32),
            pltpu.VMEM((1, NCHUNK, CH), jnp.int32),
            pltpu.VMEM((1, NCHUNK, CH), F32),
            pltpu.VMEM((NCHUNK, CH), F32),
            pltpu.VMEM((RPT,), F32),
            pltpu.VMEM_SHARED((NP,), F32),
            pltpu.SemaphoreType.DMA,
            pltpu.SemaphoreType.DMA,
        ],
    )


def _norm_body(src_h, dst_h, attr_h, dis_h, out_h, esrc, edst, eatt, dsb, ddb, nbuf, sem, gsem):
    wid = _wid()

    pltpu.async_copy(src_h.at[pl.ds(wid, 1)], esrc, sem).wait()
    pltpu.async_copy(dst_h.at[pl.ds(wid, 1)], edst, sem).wait()
    pltpu.async_copy(attr_h.at[pl.ds(wid, 1)], eatt, sem).wait()

    gd = [None, None]
    gd[0] = (
        pltpu.async_copy(dis_h.at[esrc.at[0, 0]], dsb.at[0], gsem),
        pltpu.async_copy(dis_h.at[edst.at[0, 0]], ddb.at[0], gsem),
    )
    for i in range(NCHUNK):
        cur = i % 2
        nxt = 1 - cur
        for d in gd[cur]:
            d.wait()
        if i + 1 < NCHUNK:
            gd[nxt] = (
                pltpu.async_copy(dis_h.at[esrc.at[0, i + 1]], dsb.at[nxt], gsem),
                pltpu.async_copy(dis_h.at[edst.at[0, i + 1]], ddb.at[nxt], gsem),
            )

        def sub(k, _):
            s16 = esrc[0, i, pl.ds(k * 16, 16)]
            d16 = edst[0, i, pl.ds(k * 16, 16)]
            a16 = eatt[0, i, pl.ds(k * 16, 16)]
            w16 = jnp.where(s16 != d16, a16, 0.0)
            nbuf[0, i, pl.ds(k * 16, 16)] = -(
                dsb[cur, pl.ds(k * 16, 16)] * w16 * ddb[cur, pl.ds(k * 16, 16)]
            )
            return 0

        lax.fori_loop(0, CH // 16, sub, 0)
    pltpu.sync_copy(nbuf, out_h.at[pl.ds(wid, 1)])


@functools.cache
def _norm_kernel():
    return pl.kernel(
        _norm_body,
        out_type=jax.ShapeDtypeStruct((NW, NCHUNK, CH), F32),
        mesh=_mesh(),
        compiler_params=pltpu.CompilerParams(use_tc_tiling_on_sc=False),
        scratch_types=[
            pltpu.VMEM((1, NCHUNK, CH), jnp.int32),
            pltpu.VMEM((1, NCHUNK, CH), jnp.int32),
            pltpu.VMEM((1, NCHUNK, CH), F32),
            pltpu.VMEM((2, CH), F32),
            pltpu.VMEM((2, CH), F32),
            pltpu.VMEM((1, NCHUNK, CH), F32),
            pltpu.SemaphoreType.DMA,
            pltpu.SemaphoreType.DMA,
        ],
    )


@functools.cache
def _make_prop(D):
    JV = D // 16

    def body(y_h, src_h, dst_h, nrm_h, out_h, esrc, edst, enrm, rows, zb, acc,
             sem, gsA, gsB, ssA, ssB):
        cid = lax.axis_index("c")
        sid = lax.axis_index("s")
        wid = _wid()
        gsem = [gsA, gsB]
        ssem = [ssA, ssB]

        d1 = pltpu.async_copy(src_h.at[pl.ds(wid, 1)], esrc, sem)
        d2 = pltpu.async_copy(dst_h.at[pl.ds(wid, 1)], edst, sem)
        d3 = pltpu.async_copy(nrm_h.at[pl.ds(wid, 1)], enrm, sem)

        def zero(r, _):
            for j in range(JV):
                zb[r, pl.ds(j * 16, 16)] = jnp.zeros((16,), F32)
            return 0

        lax.fori_loop(0, ZR, zero, 0)
        for k in range(RPT // ZR):
            pltpu.sync_copy(zb, acc.at[pl.ds(sid * RPT + k * ZR, ZR)])
        d1.wait()
        d2.wait()
        d3.wait()
        plsc.subcore_barrier()

        def gather(i, b):
            return pltpu.async_copy(y_h.at[esrc.at[0, i]], rows.at[b], gsem[b])

        def scatter(i, b):
            return pltpu.async_copy(rows.at[b], acc.at[edst.at[0, i]], ssem[b], add=True)

        gd = [None, None]
        sd = [None, None]
        gd[0] = gather(0, 0)
        for i in range(NCHUNK):
            cur = i % 2
            nxt = 1 - cur
            gd[cur].wait()
            if i + 1 < NCHUNK:
                if sd[nxt] is not None:
                    sd[nxt].wait()
                gd[nxt] = gather(i + 1, nxt)

            def scale(k, _):
                n16 = enrm[0, i, pl.ds(k * 16, 16)]
                for e in range(16):
                    s = n16[e]
                    r = k * 16 + e
                    for j in range(JV):
                        rows[cur, r, pl.ds(j * 16, 16)] = (
                            rows[cur, r, pl.ds(j * 16, 16)] * s
                        )
                return 0

            lax.fori_loop(0, CH // 16, scale, 0)
            sd[cur] = scatter(i, cur)
        for d in sd:
            if d is not None:
                d.wait()
        plsc.subcore_barrier()
        for k in range(RPT // ZR):
            pltpu.sync_copy(
                acc.at[pl.ds(sid * RPT + k * ZR, ZR)],
                out_h.at[pl.ds(cid * NP + sid * RPT + k * ZR, ZR)],
            )

    return pl.kernel(
        body,
        out_type=jax.ShapeDtypeStruct((2 * NP, D), F32),
        mesh=_mesh(),
        compiler_params=pltpu.CompilerParams(use_tc_tiling_on_sc=False),
        scratch_types=[
            pltpu.VMEM((1, NCHUNK, CH), jnp.int32),
            pltpu.VMEM((1, NCHUNK, CH), jnp.int32),
            pltpu.VMEM((1, NCHUNK, CH), F32),
            pltpu.VMEM((2, CH, D), F32),
            pltpu.VMEM((ZR, D), F32),
            pltpu.VMEM_SHARED((NP, D), F32),
            pltpu.SemaphoreType.DMA,
            pltpu.SemaphoreType.DMA,
            pltpu.SemaphoreType.DMA,
            pltpu.SemaphoreType.DMA,
            pltpu.SemaphoreType.DMA,
        ],
    )


BR = 2048
NBLK = NP // BR


def _row_spec(width):
    return pl.BlockSpec((BR, width), lambda i: (i, 0))


def _row_spec_off(width, off_blocks):
    return pl.BlockSpec((BR, width), lambda i: (i + off_blocks, 0))


def _full_spec(r, c):
    return pl.BlockSpec((r, c), lambda i: (0, 0))


def _l1pre_body(x_r, degp_r, w1_r, w2_r, w02_r, b_r, y1_r, y2_r, a0_r, dis_r):
    xb = x_r[...]
    y1_r[...] = jnp.dot(xb, w1_r[...], preferred_element_type=F32)
    y2_r[...] = jnp.dot(xb, w2_r[...], preferred_element_type=F32)
    a0_r[...] = jnp.dot(xb, w02_r[...], preferred_element_type=F32) + b_r[...]
    deg = degp_r[0:1, :] + degp_r[1:2, :]
    dis_r[...] = jnp.where(deg > 0, lax.rsqrt(jnp.where(deg > 0, deg, 1.0)), 0.0)


def _tc_l1pre(xp, degp, w1, w2, w02, b1):
    return pl.pallas_call(
        _l1pre_body,
        grid=(NBLK,),
        in_specs=[
            _row_spec(128),
            pl.BlockSpec((2, BR), lambda i: (0, i)),
            _full_spec(128, 64),
            _full_spec(128, 64),
            _full_spec(128, 64),
            _full_spec(1, 64),
        ],
        out_specs=[
            _row_spec(64),
            _row_spec(64),
            _row_spec(64),
            pl.BlockSpec((1, BR), lambda i: (0, i)),
        ],
        out_shape=[
            jax.ShapeDtypeStruct((NP, 64), F32),
            jax.ShapeDtypeStruct((NP, 64), F32),
            jax.ShapeDtypeStruct((NP, 64), F32),
            jax.ShapeDtypeStruct((1, NP), F32),
        ],
    )(xp, degp, w1, w2, w02, b1)


def _comb_body(pa_r, pb_r, c_r):
    c_r[...] = pa_r[...] + pb_r[...]


def _tc_comb(pp):
    return pl.pallas_call(
        _comb_body,
        grid=(NBLK,),
        in_specs=[_row_spec(64), _row_spec_off(64, NBLK)],
        out_specs=_row_spec(64),
        out_shape=jax.ShapeDtypeStruct((NP, 64), F32),
    )(pp, pp)


def _h1_body(a0_r, pa_r, pb_r, qa_r, qb_r, h1_r):
    h1_r[...] = jnp.maximum(
        a0_r[...] + pa_r[...] + pb_r[...] + 2.0 * (qa_r[...] + qb_r[...]), 0.0
    )


def _tc_h1(a0, p1p, qp):
    return pl.pallas_call(
        _h1_body,
        grid=(NBLK,),
        in_specs=[
            _row_spec(64),
            _row_spec(64),
            _row_spec_off(64, NBLK),
            _row_spec(64),
            _row_spec_off(64, NBLK),
        ],
        out_specs=_row_spec(64),
        out_shape=jax.ShapeDtypeStruct((NP, 64), F32),
    )(a0, p1p, p1p, qp, qp)


def _make_mid(dout):
    def body(h_r, ta_r, tb_r, w02_r, w1_r, b_r, t1c_r, z_r):
        t1 = ta_r[...] + tb_r[...]
        t1c_r[...] = t1
        z_r[...] = (
            jnp.dot(h_r[...], w02_r[...], preferred_element_type=F32)
            + jnp.dot(t1, w1_r[...], preferred_element_type=F32)
            + b_r[...]
        )

    def call(h, t1p, w02, w1, b):
        return pl.pallas_call(
            body,
            grid=(NBLK,),
            in_specs=[
                _row_spec(64),
                _row_spec(64),
                _row_spec_off(64, NBLK),
                _full_spec(64, dout),
                _full_spec(64, dout),
                _full_spec(1, dout),
            ],
            out_specs=[_row_spec(64), _row_spec(dout)],
            out_shape=[
                jax.ShapeDtypeStruct((NP, 64), F32),
                jax.ShapeDtypeStruct((NP, dout), F32),
            ],
        )(h, t1p, t1p, w02, w1, b)

    return call


_tc_mid64 = _make_mid(64)
_tc_mid128 = _make_mid(128)


def _make_post(dout):
    def body(z_r, ta_r, tb_r, w2_r, h_r):
        t2 = ta_r[...] + tb_r[...]
        h_r[...] = jnp.maximum(
            z_r[...] + 2.0 * jnp.dot(t2, w2_r[...], preferred_element_type=F32), 0.0
        )

    def call(z, t2p, w2):
        return pl.pallas_call(
            body,
            grid=(NBLK,),
            in_specs=[
                _row_spec(dout),
                _row_spec(64),
                _row_spec_off(64, NBLK),
                _full_spec(64, dout),
            ],
            out_specs=_row_spec(dout),
            out_shape=jax.ShapeDtypeStruct((NP, dout), F32),
        )(z, t2p, t2p, w2)

    return call


_tc_post64 = _make_post(64)
_tc_post128 = _make_post(128)


def _final_body(h3_r, batch_r, wl_r, bl_r, out_r, hg_r):
    gi = lax.broadcasted_iota(jnp.int32, (GG, NP), 0)
    oh = (batch_r[...] == gi).astype(F32)
    sums = jnp.dot(oh, h3_r[...], preferred_element_type=F32)
    cnt = jnp.sum(oh, axis=1, keepdims=True)
    hg = sums / jnp.maximum(cnt, 1.0)
    hg_r[...] = hg
    out_r[...] = jnp.dot(hg, wl_r[...], preferred_element_type=F32) + bl_r[...]


def _tc_final(h3, batchp, wl, bl):
    return pl.pallas_call(
        _final_body,
        out_shape=[
            jax.ShapeDtypeStruct((GG, 10), F32),
            jax.ShapeDtypeStruct((GG, 128), F32),
        ],
    )(h3, batchp, wl, bl)


@jax.jit
def kernel(x, edge_index, edge_attr, batch, W1_0, W1_1, W1_2, b1, W2_0, W2_1, W2_2,
           b2, W3_0, W3_1, W3_2, b3, Wl, bl):
    src = edge_index[0].reshape(NW, NCHUNK, CH)
    dst = edge_index[1].reshape(NW, NCHUNK, CH)
    attr3 = edge_attr.reshape(NW, NCHUNK, CH)

    xp = jnp.pad(x, ((0, NP - NN), (0, 0)))
    batchp = jnp.pad(batch, (0, NP - NN), constant_values=-1).reshape(1, NP)

    w1_02 = W1_0 - W1_2
    w2_02 = W2_0 - W2_2
    w3_02 = W3_0 - W3_2

    prop64 = _make_prop(64)

    degp = _deg_kernel()(src, dst, attr3).reshape(2, NP)
    y1, y2, a0, dis = _tc_l1pre(xp, degp, W1_1, W1_2, w1_02, b1.reshape(1, 64))
    norm = _norm_kernel()(src, dst, attr3, dis.reshape(NP))

    p1p = prop64(y1, src, dst, norm)
    p2p = prop64(y2, src, dst, norm)
    p2c = _tc_comb(p2p)
    qp = prop64(p2c, src, dst, norm)
    h1 = _tc_h1(a0, p1p, qp)

    t1p = prop64(h1, src, dst, norm)
    t1c, z2 = _tc_mid64(h1, t1p, w2_02, W2_1, b2.reshape(1, 64))
    t2p = prop64(t1c, src, dst, norm)
    h2 = _tc_post64(z2, t2p, W2_2)

    u1p = prop64(h2, src, dst, norm)
    u1c, z3 = _tc_mid128(h2, u1p, w3_02, W3_1, b3.reshape(1, 128))
    u2p = prop64(u1c, src, dst, norm)
    h3 = _tc_post128(z3, u2p, W3_2)

    out, hg = _tc_final(h3, batchp, Wl, bl.reshape(1, 10))
    return (out, hg)
